# Initial kernel scaffold; baseline (speedup 1.0000x reference)
import jax
import jax.numpy as jnp
from jax.experimental import pallas as pl


def kernel(system_emb, W_proj, b_proj):
    raise NotImplementedError("write your pallas kernel here")


# speedup vs baseline: 1.0208x; 1.0208x over previous
import jax
import jax.numpy as jnp
from jax.experimental import pallas as pl
from jax.experimental.pallas import tpu as pltpu

_N_LAYER = 24
_N_HEAD = 16
_HEAD_SIZE = 64
_TOTAL = 3072
_OUT = 1024
_T = 4096
_TCHUNK = 512
_NT = _T // _TCHUNK


def _mean_body(x_ref, out_ref):
    i = pl.program_id(0)

    @pl.when(i == 0)
    def _():
        out_ref[...] = jnp.zeros_like(out_ref)

    out_ref[...] += jnp.sum(x_ref[0], axis=0, keepdims=True)


def _proj_body(v_ref, w_ref, b_ref, out_ref):
    v = v_ref[...] * (1.0 / _T)
    w = w_ref[0]
    acc = jax.lax.dot_general(
        v, w, (((1,), (1,)), ((), ())), preferred_element_type=jnp.float32
    )
    out_ref[0] = acc + b_ref[0]


def kernel(system_emb, W_proj, b_proj):
    sums = pl.pallas_call(
        _mean_body,
        grid=(_NT,),
        in_specs=[pl.BlockSpec((1, _TCHUNK, _TOTAL), lambda i: (0, i, 0))],
        out_specs=pl.BlockSpec((1, _TOTAL), lambda i: (0, 0)),
        out_shape=jax.ShapeDtypeStruct((1, _TOTAL), jnp.float32),
        compiler_params=pltpu.CompilerParams(dimension_semantics=("arbitrary",)),
    )(system_emb)

    out = pl.pallas_call(
        _proj_body,
        grid=(_N_LAYER,),
        in_specs=[
            pl.BlockSpec((1, _TOTAL), lambda l: (0, 0)),
            pl.BlockSpec((1, _OUT, _TOTAL), lambda l: (l, 0, 0)),
            pl.BlockSpec((1, 1, _OUT), lambda l: (l, 0, 0)),
        ],
        out_specs=pl.BlockSpec((1, 1, _OUT), lambda l: (l, 0, 0)),
        out_shape=jax.ShapeDtypeStruct((_N_LAYER, 1, _OUT), jnp.float32),
        compiler_params=pltpu.CompilerParams(dimension_semantics=("arbitrary",)),
    )(sums, W_proj, b_proj.reshape(_N_LAYER, 1, _OUT))

    return out.reshape(_N_LAYER, _N_HEAD, _HEAD_SIZE)
